# baseline (speedup 1.0000x reference)
import functools

import numpy as np

import jax
import jax.numpy as jnp
from jax import lax
from jax.experimental import pallas as pl
from jax.experimental.pallas import tpu as pltpu
from jax.experimental.pallas import tpu_sc as plsc

V = 1000000
D = 64
B = 16384
K = 20

NC = 2
NS = 16
NW = NC * NS
BPW = B // NW
CHUNK = 128
NCH = BPW // CHUNK
GRPS = CHUNK // 16

TBLK = 32768
HB = TBLK // 2
NQ = (V + TBLK - 1) // TBLK
VROWS = NQ * TBLK


def _tr_body(x_ref, o_ref):
    t = x_ref[...].T
    o_ref[...] = jnp.concatenate([t[:HB], t[HB:]], axis=1)


_tc_transpose = pl.pallas_call(
    _tr_body,
    grid=(NQ,),
    in_specs=[pl.BlockSpec((D, TBLK), lambda q: (0, q))],
    out_specs=pl.BlockSpec((HB, 2 * D), lambda q: (q, 0)),
    out_shape=jax.ShapeDtypeStruct((VROWS // 2, 2 * D), jnp.float32),
)


def _remap(r):
    p = r % TBLK
    return (r // TBLK) * TBLK + 2 * (p % HB) + (p // HB)


CW = 1024
SCCH = 17
SCCOLS = SCCH * NW * CW
NQT = (V - SCCOLS + TBLK - 1) // TBLK
VROWST = NQT * TBLK


_tc_transpose_tail = pl.pallas_call(
    _tr_body,
    grid=(NQT,),
    in_specs=[pl.BlockSpec((D, TBLK), lambda q: (0, q + SCCOLS // TBLK))],
    out_specs=pl.BlockSpec((HB, 2 * D), lambda q: (q, 0)),
    out_shape=jax.ShapeDtypeStruct((VROWST // 2, 2 * D), jnp.float32),
)


def _remap_sc(r):
    p = r % CW
    return (r // CW) * CW + 2 * (p % (CW // 2)) + (p // (CW // 2))


def _remap_tail(r):
    return _remap(r - SCCOLS)


def _dots16(buf, vbuf, scr, g):
    def row(j, _):
        r = g * 16 + j
        p = buf[r, pl.ds(0, 16)] * vbuf[r, pl.ds(0, 16)]
        p = p + buf[r, pl.ds(16, 16)] * vbuf[r, pl.ds(16, 16)]
        p = p + buf[r, pl.ds(32, 16)] * vbuf[r, pl.ds(32, 16)]
        p = p + buf[r, pl.ds(48, 16)] * vbuf[r, pl.ds(48, 16)]
        plsc.store_scatter(scr, [lax.iota(jnp.int32, 16) * 16 + j], p)
        return 0

    lax.fori_loop(0, 16, row, 0)

    def srow(i, a):
        return a + scr[pl.ds(i * 16, 16)]

    return lax.fori_loop(0, 16, srow, jnp.zeros((16,), jnp.float32))


_mesh = plsc.VectorSubcoreMesh(core_axis_name="c", subcore_axis_name="s")
_SC_PARAMS = pltpu.CompilerParams(
    needs_layout_passes=False, use_tc_tiling_on_sc=False
)


@functools.partial(
    pl.kernel,
    mesh=_mesh,
    compiler_params=pltpu.CompilerParams(
        needs_layout_passes=False, use_tc_tiling_on_sc=True
    ),
    out_type=jax.ShapeDtypeStruct((SCCOLS // 2, 2 * D), jnp.float32),
    scratch_types=[
        pltpu.VMEM((D, CW), jnp.float32),
        pltpu.VMEM((CW // 2, 2 * D), jnp.float32),
        pltpu.SemaphoreType.DMA,
    ],
)
def _sc_conv(xt, out, buf, obuf, sem):
    wid = lax.axis_index("s") * NC + lax.axis_index("c")

    def ch(i, _):
        c = wid * SCCH + i
        pltpu.sync_copy(xt.at[:, pl.ds(c * CW, CW)], buf)

        def dl(d, _):
            def grp(g, _):
                vals = buf[d, pl.ds(g * 16, 16)]
                row0 = (g * 16) % (CW // 2)
                col = ((g * 16) // (CW // 2)) * D + d
                plsc.store_scatter(
                    obuf,
                    [lax.iota(jnp.int32, 16) + row0,
                     jnp.full((16,), col, jnp.int32)],
                    vals,
                )
                return 0

            lax.fori_loop(0, CW // 16, grp, 0)
            return 0

        lax.fori_loop(0, D, dl, 0)
        pltpu.sync_copy(obuf, out.at[pl.ds(c * (CW // 2), CW // 2)])
        return 0

    lax.fori_loop(0, SCCH, ch, 0)


@functools.partial(
    pl.kernel,
    mesh=_mesh,
    compiler_params=_SC_PARAMS,
    out_type=(
        jax.ShapeDtypeStruct((B, D), jnp.float32),
        jax.ShapeDtypeStruct((B, D), jnp.float32),
    ),
    scratch_types=[
        pltpu.VMEM((CHUNK,), jnp.int32),
        pltpu.VMEM((K, CHUNK), jnp.int32),
        pltpu.VMEM((CHUNK, D), jnp.float32),
        pltpu.VMEM((CHUNK, D), jnp.float32),
        pltpu.VMEM((CHUNK, D), jnp.float32),
        pltpu.VMEM((CHUNK,), jnp.int32),
        pltpu.VMEM_SHARED((NS * CHUNK, D), jnp.float32),
        pltpu.SemaphoreType.DMA,
        pltpu.SemaphoreType.DMA,
    ],
)
def _sc_stage(ctx, negt, oemb, uout, sout,
              ctxi, negi, ubuf, nbuf0, nbuf1, idxv, shared, sem, sem2):
    cid = lax.axis_index("c")
    sid = lax.axis_index("s")
    wid = sid * NC + cid

    def ib(g, _):
        idxv[pl.ds(g * 16, 16)] = (
            lax.iota(jnp.int32, 16) + (sid * CHUNK + g * 16)
        )
        return 0

    lax.fori_loop(0, GRPS, ib, 0)

    nbufs = (nbuf0, nbuf1)
    sems = (sem, sem2)
    for c in range(NCH):
        off = wid * BPW + c * CHUNK
        pltpu.sync_copy(ctx.at[pl.ds(off, CHUNK)], ctxi)
        pltpu.sync_copy(negt.at[:, pl.ds(off, CHUNK)], negi)
        pltpu.sync_copy(oemb.at[ctxi], ubuf)
        pltpu.sync_copy(ubuf, uout.at[pl.ds(off, CHUNK)])
        cp = pltpu.async_copy(oemb.at[negi.at[0]], nbufs[0], sems[0])
        for k in range(K):
            if k + 1 < K:
                nxt = pltpu.async_copy(
                    oemb.at[negi.at[k + 1]], nbufs[(k + 1) % 2],
                    sems[(k + 1) % 2],
                )
            cp.wait()
            pltpu.sync_copy(nbufs[k % 2], shared.at[idxv], add=(k > 0))
            if k + 1 < K:
                cp = nxt
        pltpu.sync_copy(
            shared.at[pl.ds(sid * CHUNK, CHUNK)], sout.at[pl.ds(off, CHUNK)]
        )


@functools.partial(
    pl.kernel,
    mesh=_mesh,
    compiler_params=_SC_PARAMS,
    out_type=(
        jax.ShapeDtypeStruct((B,), jnp.float32),
        jax.ShapeDtypeStruct((B,), jnp.float32),
    ),
    scratch_types=[
        pltpu.VMEM((CHUNK,), jnp.int32),
        pltpu.VMEM((CHUNK,), jnp.int32),
        pltpu.VMEM((CHUNK,), jnp.int32),
        pltpu.VMEM((CHUNK, D), jnp.float32),
        pltpu.VMEM((CHUNK, D), jnp.float32),
        pltpu.VMEM((CHUNK, D), jnp.float32),
        pltpu.VMEM((CHUNK, D), jnp.float32),
        pltpu.VMEM((256,), jnp.float32),
        pltpu.VMEM((CHUNK,), jnp.float32),
        pltpu.VMEM((CHUNK,), jnp.float32),
        pltpu.SemaphoreType.DMA,
    ],
)
def _sc_dots2(tgta_h, tgtb_h, flg_h, uin, sin, iemba, iembb, pdot, ndot,
              tgta, tgtb, flg, vba, vbb, ub2, sb2, scr, pv, nv, sem):
    wid = lax.axis_index("s") * NC + lax.axis_index("c")
    for c in range(NCH):
        off = wid * BPW + c * CHUNK
        pltpu.sync_copy(tgta_h.at[pl.ds(off, CHUNK)], tgta)
        pltpu.sync_copy(tgtb_h.at[pl.ds(off, CHUNK)], tgtb)
        pltpu.sync_copy(flg_h.at[pl.ds(off, CHUNK)], flg)
        ucp = pltpu.async_copy(uin.at[pl.ds(off, CHUNK)], ub2, sem)
        scp = pltpu.async_copy(sin.at[pl.ds(off, CHUNK)], sb2, sem)
        acp = pltpu.async_copy(iemba.at[tgta], vba, sem)
        pltpu.async_copy(iembb.at[tgtb], vbb, sem).wait()
        acp.wait()
        scp.wait()
        ucp.wait()

        def grp(g, _):
            f = flg[pl.ds(g * 16, 16)] > 0
            pa = _dots16(ub2, vba, scr, g)
            pb = _dots16(ub2, vbb, scr, g)
            pv[pl.ds(g * 16, 16)] = jnp.where(f, pa, pb)
            na = _dots16(sb2, vba, scr, g)
            nb = _dots16(sb2, vbb, scr, g)
            nv[pl.ds(g * 16, 16)] = jnp.where(f, na, nb)
            return 0

        lax.fori_loop(0, GRPS, grp, 0)

        pltpu.sync_copy(pv, pdot.at[pl.ds(off, CHUNK)])
        pltpu.sync_copy(nv, ndot.at[pl.ds(off, CHUNK)])


def _tc_body(p_ref, n_ref, o_ref):
    p = p_ref[...]
    n = n_ref[...]
    lp = jnp.minimum(p, 0.0) - jnp.log1p(jnp.exp(-jnp.abs(p)))
    ln = jnp.minimum(-n, 0.0) - jnp.log1p(jnp.exp(-jnp.abs(n)))
    o_ref[0, 0] = -jnp.sum(lp + ln) * (1.0 / B)


_tc_loss = pl.pallas_call(
    _tc_body,
    out_shape=jax.ShapeDtypeStruct((1, 1), jnp.float32),
    out_specs=pl.BlockSpec(memory_space=pltpu.SMEM),
)


def kernel(target, context, negative_word_batch, input_emb, output_emb):
    neg_t = jnp.transpose(negative_word_batch)
    xt_in = input_emb.T
    iemb_sc = _sc_conv(xt_in).reshape(SCCOLS, D)
    oemb = _tc_transpose(output_emb.T).reshape(VROWS, D)
    u_rows, s_rows = _sc_stage(_remap(context), _remap(neg_t), oemb)
    iemb_tc = _tc_transpose_tail(xt_in).reshape(VROWST, D)
    in_sc = target < SCCOLS
    tgta = _remap_sc(jnp.where(in_sc, target, 0))
    tgtb = _remap_tail(jnp.where(in_sc, SCCOLS, target))
    pdot, ndot = _sc_dots2(
        tgta, tgtb, in_sc.astype(jnp.int32), u_rows, s_rows, iemb_sc, iemb_tc
    )
    out = _tc_loss(pdot.reshape(128, 128), ndot.reshape(128, 128))
    return out.reshape(())


# speedup vs baseline: 2.7750x; 2.7750x over previous
import functools

import numpy as np

import jax
import jax.numpy as jnp
from jax import lax
from jax.experimental import pallas as pl
from jax.experimental.pallas import tpu as pltpu
from jax.experimental.pallas import tpu_sc as plsc

V = 1000000
D = 64
B = 16384
K = 20

NC = 2
NS = 16
NW = NC * NS
BPW = B // NW
CHUNK = 128
NCH = BPW // CHUNK
GRPS = CHUNK // 16

TBLK = 32768
HB = TBLK // 2
NQ = (V + TBLK - 1) // TBLK
VROWS = NQ * TBLK


def _tr_body(x_ref, o_ref):
    t = x_ref[...].T
    o_ref[...] = jnp.concatenate([t[:HB], t[HB:]], axis=1)


_tc_transpose = pl.pallas_call(
    _tr_body,
    grid=(NQ,),
    in_specs=[pl.BlockSpec((D, TBLK), lambda q: (0, q))],
    out_specs=pl.BlockSpec((HB, 2 * D), lambda q: (q, 0)),
    out_shape=jax.ShapeDtypeStruct((VROWS // 2, 2 * D), jnp.float32),
)


def _remap(r):
    p = r % TBLK
    return (r // TBLK) * TBLK + 2 * (p % HB) + (p // HB)


def _dots16(buf, vbuf, scr, g):
    def row(j, _):
        r = g * 16 + j
        p = buf[r, pl.ds(0, 16)] * vbuf[r, pl.ds(0, 16)]
        p = p + buf[r, pl.ds(16, 16)] * vbuf[r, pl.ds(16, 16)]
        p = p + buf[r, pl.ds(32, 16)] * vbuf[r, pl.ds(32, 16)]
        p = p + buf[r, pl.ds(48, 16)] * vbuf[r, pl.ds(48, 16)]
        plsc.store_scatter(scr, [lax.iota(jnp.int32, 16) * 16 + j], p)
        return 0

    lax.fori_loop(0, 16, row, 0)

    def srow(i, a):
        return a + scr[pl.ds(i * 16, 16)]

    return lax.fori_loop(0, 16, srow, jnp.zeros((16,), jnp.float32))


_mesh = plsc.VectorSubcoreMesh(core_axis_name="c", subcore_axis_name="s")
_SC_PARAMS = pltpu.CompilerParams(
    needs_layout_passes=False, use_tc_tiling_on_sc=False
)


@functools.partial(
    pl.kernel,
    mesh=_mesh,
    compiler_params=_SC_PARAMS,
    out_type=(
        jax.ShapeDtypeStruct((B, D), jnp.float32),
        jax.ShapeDtypeStruct((B, D), jnp.float32),
    ),
    scratch_types=[
        pltpu.VMEM((CHUNK,), jnp.int32),
        pltpu.VMEM((K, CHUNK), jnp.int32),
        pltpu.VMEM((CHUNK, D), jnp.float32),
        pltpu.VMEM((CHUNK, D), jnp.float32),
        pltpu.VMEM((CHUNK, D), jnp.float32),
        pltpu.VMEM((CHUNK,), jnp.int32),
        pltpu.VMEM_SHARED((NS * CHUNK, D), jnp.float32),
        pltpu.SemaphoreType.DMA,
        pltpu.SemaphoreType.DMA,
    ],
)
def _sc_stage(ctx, negt, oemb, uout, sout,
              ctxi, negi, ubuf, nbuf0, nbuf1, idxv, shared, sem, sem2):
    cid = lax.axis_index("c")
    sid = lax.axis_index("s")
    wid = sid * NC + cid

    def ib(g, _):
        idxv[pl.ds(g * 16, 16)] = (
            lax.iota(jnp.int32, 16) + (sid * CHUNK + g * 16)
        )
        return 0

    lax.fori_loop(0, GRPS, ib, 0)

    nbufs = (nbuf0, nbuf1)
    sems = (sem, sem2)
    for c in range(NCH):
        off = wid * BPW + c * CHUNK
        pltpu.sync_copy(ctx.at[pl.ds(off, CHUNK)], ctxi)
        pltpu.sync_copy(negt.at[:, pl.ds(off, CHUNK)], negi)
        pltpu.sync_copy(oemb.at[ctxi], ubuf)
        pltpu.sync_copy(ubuf, uout.at[pl.ds(off, CHUNK)])
        cp = pltpu.async_copy(oemb.at[negi.at[0]], nbufs[0], sems[0])
        for k in range(K):
            if k + 1 < K:
                nxt = pltpu.async_copy(
                    oemb.at[negi.at[k + 1]], nbufs[(k + 1) % 2],
                    sems[(k + 1) % 2],
                )
            cp.wait()
            pltpu.sync_copy(nbufs[k % 2], shared.at[idxv], add=(k > 0))
            if k + 1 < K:
                cp = nxt
        pltpu.sync_copy(
            shared.at[pl.ds(sid * CHUNK, CHUNK)], sout.at[pl.ds(off, CHUNK)]
        )


@functools.partial(
    pl.kernel,
    mesh=_mesh,
    compiler_params=_SC_PARAMS,
    out_type=(
        jax.ShapeDtypeStruct((B,), jnp.float32),
        jax.ShapeDtypeStruct((B,), jnp.float32),
    ),
    scratch_types=[
        pltpu.VMEM((CHUNK,), jnp.int32),
        pltpu.VMEM((CHUNK, D), jnp.float32),
        pltpu.VMEM((CHUNK, D), jnp.float32),
        pltpu.VMEM((CHUNK, D), jnp.float32),
        pltpu.VMEM((256,), jnp.float32),
        pltpu.VMEM((CHUNK,), jnp.float32),
        pltpu.VMEM((CHUNK,), jnp.float32),
        pltpu.SemaphoreType.DMA,
    ],
)
def _sc_dots2(tgt, uin, sin, iemb, pdot, ndot,
              tgti, vbuf, ub2, sb2, scr, pv, nv, sem):
    wid = lax.axis_index("s") * NC + lax.axis_index("c")
    for c in range(NCH):
        off = wid * BPW + c * CHUNK
        pltpu.sync_copy(tgt.at[pl.ds(off, CHUNK)], tgti)
        ucp = pltpu.async_copy(uin.at[pl.ds(off, CHUNK)], ub2, sem)
        scp = pltpu.async_copy(sin.at[pl.ds(off, CHUNK)], sb2, sem)
        pltpu.async_copy(iemb.at[tgti], vbuf, sem).wait()
        scp.wait()
        ucp.wait()

        def grp(g, _):
            pv[pl.ds(g * 16, 16)] = _dots16(ub2, vbuf, scr, g)
            nv[pl.ds(g * 16, 16)] = _dots16(sb2, vbuf, scr, g)
            return 0

        lax.fori_loop(0, GRPS, grp, 0)

        pltpu.sync_copy(pv, pdot.at[pl.ds(off, CHUNK)])
        pltpu.sync_copy(nv, ndot.at[pl.ds(off, CHUNK)])


def _tc_body(p_ref, n_ref, o_ref):
    p = p_ref[...]
    n = n_ref[...]
    lp = jnp.minimum(p, 0.0) - jnp.log1p(jnp.exp(-jnp.abs(p)))
    ln = jnp.minimum(-n, 0.0) - jnp.log1p(jnp.exp(-jnp.abs(n)))
    o_ref[0, 0] = -jnp.sum(lp + ln) * (1.0 / B)


_tc_loss = pl.pallas_call(
    _tc_body,
    out_shape=jax.ShapeDtypeStruct((1, 1), jnp.float32),
    out_specs=pl.BlockSpec(memory_space=pltpu.SMEM),
)


def kernel(target, context, negative_word_batch, input_emb, output_emb):
    neg_t = jnp.transpose(negative_word_batch)
    oemb = _tc_transpose(output_emb.T).reshape(VROWS, D)
    u_rows, s_rows = _sc_stage(_remap(context), _remap(neg_t), oemb)
    iemb = _tc_transpose(input_emb.T).reshape(VROWS, D)
    pdot, ndot = _sc_dots2(_remap(target), u_rows, s_rows, iemb)
    out = _tc_loss(pdot.reshape(128, 128), ndot.reshape(128, 128))
    return out.reshape(())
